# baseline (speedup 1.0000x reference)
import functools

import numpy as np

import jax
import jax.numpy as jnp
from jax import lax
from jax.experimental import pallas as pl
from jax.experimental.pallas import tpu as pltpu
from jax.experimental.pallas import tpu_sc as plsc

N = 10000
E = 320000
F_IN = 128
H = 16
C = 3

NC = 2
NS = 16
L = 16
NW = NC * NS

CHUNK = 128
E_PAD = 327680
EROWS = E_PAD // CHUNK
CH_PER_W = EROWS // NW
NBUF = 8
N_TBL = 10048
TRS = N_TBL // NS
PR = N_TBL * H // 128
NROWS = N * H // 128
N_ACC = 10240
ACC_PR = N_ACC * H // 128
RPS = N_ACC // NS

_PAD_EDGES = np.broadcast_to(
    np.stack([(np.arange(240) * 41 % N).astype(np.int32),
              (N + np.arange(240)).astype(np.int32)])[:, None, :],
    (2, NW, 240)).copy()

_mesh = plsc.VectorSubcoreMesh(core_axis_name="c", subcore_axis_name="s")
_sc_params = pltpu.CompilerParams(use_tc_tiling_on_sc=False)


@functools.partial(
    pl.kernel,
    out_type=jax.ShapeDtypeStruct((NC, N_ACC, L), jnp.float32),
    mesh=_mesh,
    compiler_params=_sc_params,
    scratch_types=[
        pltpu.VMEM((CH_PER_W, CHUNK), jnp.int32),
        pltpu.VMEM((CHUNK, L), jnp.float32),
        pltpu.VMEM((RPS, L), jnp.float32),
        pltpu.VMEM_SHARED((N_ACC, L), jnp.float32),
    ] + [pltpu.SemaphoreType.DMA] * NBUF,
)
def _sc_degree(dst_hbm, out_hbm, dst_v, ones_v, stage_v, acc_sh, *ssem):
    cid = lax.axis_index("c")
    sid = lax.axis_index("s")
    wid = cid * NS + sid

    zrow = jnp.zeros((L,), jnp.float32)
    orow = jnp.ones((L,), jnp.float32)

    @pl.loop(0, RPS)
    def _(i):
        stage_v[i, :] = zrow

    @pl.loop(0, CHUNK)
    def _(i):
        ones_v[i, :] = orow

    pltpu.sync_copy(stage_v, acc_sh.at[pl.ds(sid * RPS, RPS)])
    pltpu.sync_copy(dst_hbm.at[pl.ds(wid * CH_PER_W, CH_PER_W)], dst_v)
    plsc.subcore_barrier()

    def scat(j, b):
        return pltpu.make_async_copy(ones_v, acc_sh.at[dst_v.at[j]], ssem[b])

    for b in range(NBUF):
        scat(b, b).start(add=True)

    @pl.loop(0, CH_PER_W - NBUF, step=NBUF)
    def _(jj):
        for b in range(NBUF):
            scat(jj + b, b).wait()
            scat(jj + NBUF + b, b).start(add=True)

    for b in range(NBUF):
        scat(0, b).wait()

    plsc.subcore_barrier()
    pltpu.sync_copy(
        acc_sh.at[pl.ds(sid * RPS, RPS)],
        out_hbm.at[cid, pl.ds(sid * RPS, RPS)],
    )


@functools.partial(
    pl.kernel,
    out_type=jax.ShapeDtypeStruct((NC, N_ACC, H), jnp.float32),
    mesh=_mesh,
    compiler_params=_sc_params,
    scratch_types=[
        pltpu.VMEM((CH_PER_W, CHUNK), jnp.int32),
        pltpu.VMEM((CH_PER_W, CHUNK), jnp.int32),
        pltpu.VMEM((RPS, H), jnp.float32),
        pltpu.VMEM_SHARED((N_ACC, H), jnp.float32),
        pltpu.VMEM_SHARED((N_TBL, H), jnp.float32),
    ] + [pltpu.VMEM((CHUNK, H), jnp.float32)] * NBUF
      + [pltpu.SemaphoreType.DMA] * (2 * NBUF),
)
def _sc_aggregate(table_hbm, src_hbm, dst_hbm, out_hbm,
                  src_v, dst_v, stage_v, acc_sh, table_sh, *bufs_sems):
    bufs = bufs_sems[:NBUF]
    gsem = bufs_sems[NBUF:2 * NBUF]
    ssem = bufs_sems[2 * NBUF:]

    cid = lax.axis_index("c")
    sid = lax.axis_index("s")
    wid = cid * NS + sid

    zrow = jnp.zeros((L,), jnp.float32)

    @pl.loop(0, RPS)
    def _(i):
        stage_v[i, :] = zrow

    pltpu.sync_copy(stage_v, acc_sh.at[pl.ds(sid * RPS, RPS)])
    pltpu.sync_copy(table_hbm.at[pl.ds(sid * TRS, TRS)],
                    table_sh.at[pl.ds(sid * TRS, TRS)])
    pltpu.sync_copy(src_hbm.at[pl.ds(wid * CH_PER_W, CH_PER_W)], src_v)
    pltpu.sync_copy(dst_hbm.at[pl.ds(wid * CH_PER_W, CH_PER_W)], dst_v)
    plsc.subcore_barrier()

    def gat(j, b):
        return pltpu.make_async_copy(
            table_sh.at[src_v.at[j]], bufs[b], gsem[b])

    def scat(j, b):
        return pltpu.make_async_copy(bufs[b], acc_sh.at[dst_v.at[j]], ssem[b])

    for b in range(NBUF):
        gat(b, b).start()

    @pl.loop(0, CH_PER_W - NBUF, step=NBUF)
    def _(jj):
        for b in range(NBUF):
            gat(jj + b, b).wait()
            scat(jj + b, b).start(add=True)
        for b in range(NBUF):
            scat(jj + b, b).wait()
            gat(jj + NBUF + b, b).start()

    for b in range(NBUF):
        gat(CH_PER_W - NBUF + b, b).wait()
        scat(CH_PER_W - NBUF + b, b).start(add=True)
    for b in range(NBUF):
        scat(0, b).wait()

    plsc.subcore_barrier()
    pltpu.sync_copy(
        acc_sh.at[pl.ds(sid * RPS, RPS)],
        out_hbm.at[cid, pl.ds(sid * RPS, RPS)],
    )


def _mm1_body(xr_ref, w_ref, o_ref):
    o_ref[...] = jnp.dot(xr_ref[...], w_ref[...],
                         preferred_element_type=jnp.float32)


def _prep_body(h1p_ref, dp_ref, h1s_ref, dis_ref):
    deg = dp_ref[0, :PR, :] + dp_ref[1, :PR, :] + 1.0
    dis = lax.rsqrt(deg)
    dis_ref[...] = dis
    h1s_ref[...] = h1p_ref[...] * dis


def _post1_body(acc_ref, h1s_ref, dis_ref, b1_ref, o_ref):
    agg = acc_ref[0, :PR, :] + acc_ref[1, :PR, :] + h1s_ref[...]
    pre = agg * dis_ref[...] + b1_ref[...]
    o_ref[...] = jnp.maximum(pre, 0.0) * dis_ref[...]


def _final_body(acc_ref, o1s_ref, dis_ref, w2_ref, b2_ref, o_ref):
    hp = (acc_ref[0, :PR, :] + acc_ref[1, :PR, :] + o1s_ref[...]) * dis_ref[...]
    o_ref[...] = jnp.dot(
        hp, w2_ref[...], preferred_element_type=jnp.float32) + b2_ref[...]


def kernel(x, edge_index, W1, b1, W2, b2):
    dst3 = jnp.concatenate(
        [edge_index[1].reshape(NW, E // NW), _PAD_EDGES[1]],
        axis=1).reshape(EROWS, CHUNK)
    src3 = jnp.concatenate(
        [edge_index[0].reshape(NW, E // NW), _PAD_EDGES[0]],
        axis=1).reshape(EROWS, CHUNK)

    deg_parts = _sc_degree(dst3)

    xr = jnp.pad(x, ((0, N_TBL - N), (0, 0))).reshape(PR, 8 * F_IN)
    w1bd = jnp.kron(jnp.eye(128 // H, dtype=W1.dtype), W1)
    h1p = pl.pallas_call(
        _mm1_body,
        out_shape=jax.ShapeDtypeStruct((PR, 128), jnp.float32),
    )(xr, w1bd)

    h1sp, disp = pl.pallas_call(
        _prep_body,
        out_shape=[
            jax.ShapeDtypeStruct((PR, 128), jnp.float32),
            jax.ShapeDtypeStruct((PR, 128), jnp.float32),
        ],
    )(h1p, deg_parts.reshape(NC, ACC_PR, 128))

    acc1 = _sc_aggregate(h1sp.reshape(N_TBL, H), src3, dst3)

    b1t = jnp.tile(b1, 128 // H).reshape(1, 128)
    o1sp = pl.pallas_call(
        _post1_body,
        out_shape=jax.ShapeDtypeStruct((PR, 128), jnp.float32),
    )(acc1.reshape(NC, ACC_PR, 128), h1sp, disp, b1t)

    acc2 = _sc_aggregate(o1sp.reshape(N_TBL, H), src3, dst3)

    w2bd = jnp.kron(jnp.eye(128 // H, dtype=W2.dtype), W2)
    b2t = jnp.tile(b2, 128 // H).reshape(1, 8 * C)
    out24 = pl.pallas_call(
        _final_body,
        out_shape=jax.ShapeDtypeStruct((PR, 8 * C), jnp.float32),
    )(acc2.reshape(NC, ACC_PR, 128), o1sp, disp, w2bd, b2t)

    return out24.reshape(N_TBL, C)[:N, :]


# speedup vs baseline: 1.0966x; 1.0966x over previous
import functools

import numpy as np

import jax
import jax.numpy as jnp
from jax import lax
from jax.experimental import pallas as pl
from jax.experimental.pallas import tpu as pltpu
from jax.experimental.pallas import tpu_sc as plsc

N = 10000
E = 320000
F_IN = 128
H = 16
C = 3

NC = 2
NS = 16
L = 16
NW = NC * NS

CHUNK = 128
E_PAD = 327680
EROWS = E_PAD // CHUNK
CH_PER_W = EROWS // NW
NBUF = 8
N_TBL = 10048
TRS = N_TBL // NS
PR = N_TBL * H // 128
NROWS = N * H // 128
N_ACC = 10240
ACC_PR = N_ACC * H // 128
RPS = N_ACC // NS

_PAD_EDGES = np.broadcast_to(
    np.stack([(np.arange(240) * 41 % N).astype(np.int32),
              (N + np.arange(240)).astype(np.int32)])[:, None, :],
    (2, NW, 240)).copy()

_mesh = plsc.VectorSubcoreMesh(core_axis_name="c", subcore_axis_name="s")
_sc_params = pltpu.CompilerParams(use_tc_tiling_on_sc=False)


@functools.partial(
    pl.kernel,
    out_type=jax.ShapeDtypeStruct((NC, N_ACC, L), jnp.float32),
    mesh=_mesh,
    compiler_params=_sc_params,
    scratch_types=[
        pltpu.VMEM((CH_PER_W, CHUNK), jnp.int32),
        pltpu.VMEM((CHUNK, L), jnp.float32),
        pltpu.VMEM((RPS, L), jnp.float32),
        pltpu.VMEM_SHARED((N_ACC, L), jnp.float32),
    ] + [pltpu.SemaphoreType.DMA] * NBUF,
)
def _sc_degree(e_hbm, out_hbm, dst_v, ones_v, stage_v, acc_sh, *ssem):
    cid = lax.axis_index("c")
    sid = lax.axis_index("s")
    wid = cid * NS + sid

    zrow = jnp.zeros((L,), jnp.float32)
    orow = jnp.ones((L,), jnp.float32)

    @pl.loop(0, RPS)
    def _(i):
        stage_v[i, :] = zrow

    @pl.loop(0, CHUNK)
    def _(i):
        ones_v[i, :] = orow

    pltpu.sync_copy(stage_v, acc_sh.at[pl.ds(sid * RPS, RPS)])
    pltpu.sync_copy(e_hbm.at[1, pl.ds(wid * CH_PER_W, CH_PER_W)], dst_v)
    plsc.subcore_barrier()

    def scat(j, b):
        return pltpu.make_async_copy(ones_v, acc_sh.at[dst_v.at[j]], ssem[b])

    for b in range(NBUF):
        scat(b, b).start(add=True)

    @pl.loop(0, CH_PER_W - NBUF, step=NBUF)
    def _(jj):
        for b in range(NBUF):
            scat(jj + b, b).wait()
            scat(jj + NBUF + b, b).start(add=True)

    for b in range(NBUF):
        scat(0, b).wait()

    plsc.subcore_barrier()
    pltpu.sync_copy(
        acc_sh.at[pl.ds(sid * RPS, RPS)],
        out_hbm.at[cid, pl.ds(sid * RPS, RPS)],
    )


@functools.partial(
    pl.kernel,
    out_type=jax.ShapeDtypeStruct((NC, N_ACC, H), jnp.float32),
    mesh=_mesh,
    compiler_params=_sc_params,
    scratch_types=[
        pltpu.VMEM((CH_PER_W, CHUNK), jnp.int32),
        pltpu.VMEM((CH_PER_W, CHUNK), jnp.int32),
        pltpu.VMEM((RPS, H), jnp.float32),
        pltpu.VMEM_SHARED((N_ACC, H), jnp.float32),
        pltpu.VMEM_SHARED((N_TBL, H), jnp.float32),
    ] + [pltpu.VMEM((CHUNK, H), jnp.float32)] * NBUF
      + [pltpu.SemaphoreType.DMA] * (2 * NBUF),
)
def _sc_aggregate(table_hbm, e_hbm, out_hbm,
                  src_v, dst_v, stage_v, acc_sh, table_sh, *bufs_sems):
    bufs = bufs_sems[:NBUF]
    gsem = bufs_sems[NBUF:2 * NBUF]
    ssem = bufs_sems[2 * NBUF:]

    cid = lax.axis_index("c")
    sid = lax.axis_index("s")
    wid = cid * NS + sid

    zrow = jnp.zeros((L,), jnp.float32)

    @pl.loop(0, RPS)
    def _(i):
        stage_v[i, :] = zrow

    pltpu.sync_copy(stage_v, acc_sh.at[pl.ds(sid * RPS, RPS)])
    pltpu.sync_copy(table_hbm.at[pl.ds(sid * TRS, TRS)],
                    table_sh.at[pl.ds(sid * TRS, TRS)])
    pltpu.sync_copy(e_hbm.at[0, pl.ds(wid * CH_PER_W, CH_PER_W)], src_v)
    pltpu.sync_copy(e_hbm.at[1, pl.ds(wid * CH_PER_W, CH_PER_W)], dst_v)
    plsc.subcore_barrier()

    def gat(j, b):
        return pltpu.make_async_copy(
            table_sh.at[src_v.at[j]], bufs[b], gsem[b])

    def scat(j, b):
        return pltpu.make_async_copy(bufs[b], acc_sh.at[dst_v.at[j]], ssem[b])

    for b in range(NBUF):
        gat(b, b).start()

    @pl.loop(0, CH_PER_W - NBUF, step=NBUF)
    def _(jj):
        for b in range(NBUF):
            gat(jj + b, b).wait()
            scat(jj + b, b).start(add=True)
        for b in range(NBUF):
            scat(jj + b, b).wait()
            gat(jj + NBUF + b, b).start()

    for b in range(NBUF):
        gat(CH_PER_W - NBUF + b, b).wait()
        scat(CH_PER_W - NBUF + b, b).start(add=True)
    for b in range(NBUF):
        scat(0, b).wait()

    plsc.subcore_barrier()
    pltpu.sync_copy(
        acc_sh.at[pl.ds(sid * RPS, RPS)],
        out_hbm.at[cid, pl.ds(sid * RPS, RPS)],
    )


def _mm1_body(xr_ref, w_ref, o_ref):
    o_ref[...] = jnp.dot(xr_ref[...], w_ref[...],
                         preferred_element_type=jnp.float32)


def _prep_body(h1p_ref, dp_ref, h1s_ref, dis_ref):
    deg = dp_ref[0, :PR, :] + dp_ref[1, :PR, :] + 1.0
    dis = lax.rsqrt(deg)
    dis_ref[...] = dis
    h1s_ref[...] = h1p_ref[...] * dis


def _post1_body(acc_ref, h1s_ref, dis_ref, b1_ref, o_ref):
    agg = acc_ref[0, :PR, :] + acc_ref[1, :PR, :] + h1s_ref[...]
    pre = agg * dis_ref[...] + b1_ref[...]
    o_ref[...] = jnp.maximum(pre, 0.0) * dis_ref[...]


def _final_body(acc_ref, o1s_ref, dis_ref, w2_ref, b2_ref, o_ref):
    hp = (acc_ref[0, :PR, :] + acc_ref[1, :PR, :] + o1s_ref[...]) * dis_ref[...]
    o_ref[...] = jnp.dot(
        hp, w2_ref[...], preferred_element_type=jnp.float32) + b2_ref[...]


def kernel(x, edge_index, W1, b1, W2, b2):
    e3 = jnp.concatenate(
        [edge_index.reshape(2, NW, E // NW), _PAD_EDGES],
        axis=2).reshape(2, EROWS, CHUNK)

    deg_parts = _sc_degree(e3)

    xr = jnp.pad(x, ((0, N_TBL - N), (0, 0))).reshape(PR, 8 * F_IN)
    w1bd = jnp.kron(jnp.eye(128 // H, dtype=W1.dtype), W1)
    h1p = pl.pallas_call(
        _mm1_body,
        out_shape=jax.ShapeDtypeStruct((PR, 128), jnp.float32),
    )(xr, w1bd)

    h1sp, disp = pl.pallas_call(
        _prep_body,
        out_shape=[
            jax.ShapeDtypeStruct((PR, 128), jnp.float32),
            jax.ShapeDtypeStruct((PR, 128), jnp.float32),
        ],
    )(h1p, deg_parts.reshape(NC, ACC_PR, 128))

    acc1 = _sc_aggregate(h1sp.reshape(N_TBL, H), e3)

    b1t = jnp.tile(b1, 128 // H).reshape(1, 128)
    o1sp = pl.pallas_call(
        _post1_body,
        out_shape=jax.ShapeDtypeStruct((PR, 128), jnp.float32),
    )(acc1.reshape(NC, ACC_PR, 128), h1sp, disp, b1t)

    acc2 = _sc_aggregate(o1sp.reshape(N_TBL, H), e3)

    w2bd = jnp.kron(jnp.eye(128 // H, dtype=W2.dtype), W2)
    b2t = jnp.tile(b2, 128 // H).reshape(1, 8 * C)
    out24 = pl.pallas_call(
        _final_body,
        out_shape=jax.ShapeDtypeStruct((PR, 8 * C), jnp.float32),
    )(acc2.reshape(NC, ACC_PR, 128), o1sp, disp, w2bd, b2t)

    return out24.reshape(N_TBL, C)[:N, :]


# speedup vs baseline: 1.0981x; 1.0014x over previous
import functools

import numpy as np

import jax
import jax.numpy as jnp
from jax import lax
from jax.experimental import pallas as pl
from jax.experimental.pallas import tpu as pltpu
from jax.experimental.pallas import tpu_sc as plsc

N = 10000
E = 320000
F_IN = 128
H = 16
C = 3

NC = 2
NS = 16
L = 16
NW = NC * NS

CHUNK = 128
E_PAD = 327680
EROWS = E_PAD // CHUNK
CH_PER_W = EROWS // NW
NBUF = 8
N_TBL = 10048
TRS = N_TBL // NS
PR = N_TBL * H // 128
NROWS = N * H // 128
N_ACC = 10240
ACC_PR = N_ACC * H // 128
RPS = N_ACC // NS

_PAD_EDGES = np.broadcast_to(
    np.stack([(np.arange(240) * 41 % N).astype(np.int32),
              (N + np.arange(240)).astype(np.int32)])[:, None, :],
    (2, NW, 240)).copy()

_mesh = plsc.VectorSubcoreMesh(core_axis_name="c", subcore_axis_name="s")
_sc_params = pltpu.CompilerParams(use_tc_tiling_on_sc=False)


@functools.partial(
    pl.kernel,
    out_type=jax.ShapeDtypeStruct((NC, N_ACC, L), jnp.float32),
    mesh=_mesh,
    compiler_params=_sc_params,
    scratch_types=[
        pltpu.VMEM((CH_PER_W, CHUNK), jnp.int32),
        pltpu.VMEM((CHUNK, L), jnp.float32),
        pltpu.VMEM((RPS, L), jnp.float32),
        pltpu.VMEM_SHARED((N_ACC, L), jnp.float32),
    ] + [pltpu.SemaphoreType.DMA] * NBUF,
)
def _sc_degree(e_hbm, out_hbm, dst_v, ones_v, stage_v, acc_sh, *ssem):
    cid = lax.axis_index("c")
    sid = lax.axis_index("s")
    wid = cid * NS + sid

    zrow = jnp.zeros((L,), jnp.float32)
    orow = jnp.ones((L,), jnp.float32)

    @pl.loop(0, RPS)
    def _(i):
        stage_v[i, :] = zrow

    @pl.loop(0, CHUNK)
    def _(i):
        ones_v[i, :] = orow

    pltpu.sync_copy(stage_v, acc_sh.at[pl.ds(sid * RPS, RPS)])
    pltpu.sync_copy(e_hbm.at[1, pl.ds(wid * CH_PER_W, CH_PER_W)], dst_v)
    plsc.subcore_barrier()

    def scat(j, b):
        return pltpu.make_async_copy(ones_v, acc_sh.at[dst_v.at[j]], ssem[b])

    for b in range(NBUF):
        scat(b, b).start(add=True)

    @pl.loop(0, CH_PER_W - NBUF, step=NBUF)
    def _(jj):
        for b in range(NBUF):
            scat(jj + b, b).wait()
            scat(jj + NBUF + b, b).start(add=True)

    for b in range(NBUF):
        scat(0, b).wait()

    plsc.subcore_barrier()
    pltpu.sync_copy(
        acc_sh.at[pl.ds(sid * RPS, RPS)],
        out_hbm.at[cid, pl.ds(sid * RPS, RPS)],
    )


@functools.partial(
    pl.kernel,
    out_type=jax.ShapeDtypeStruct((NC, N_ACC, H), jnp.float32),
    mesh=_mesh,
    compiler_params=_sc_params,
    scratch_types=[
        pltpu.VMEM((CH_PER_W, CHUNK), jnp.int32),
        pltpu.VMEM((CH_PER_W, CHUNK), jnp.int32),
        pltpu.VMEM((RPS, H), jnp.float32),
        pltpu.VMEM_SHARED((N_ACC, H), jnp.float32),
        pltpu.VMEM_SHARED((N_TBL, H), jnp.float32),
    ] + [pltpu.VMEM((CHUNK, H), jnp.float32)] * NBUF
      + [pltpu.SemaphoreType.DMA] * (2 * NBUF),
)
def _sc_aggregate(table_hbm, e_hbm, out_hbm,
                  src_v, dst_v, stage_v, acc_sh, table_sh, *bufs_sems):
    bufs = bufs_sems[:NBUF]
    gsem = bufs_sems[NBUF:2 * NBUF]
    ssem = bufs_sems[2 * NBUF:]

    cid = lax.axis_index("c")
    sid = lax.axis_index("s")
    wid = cid * NS + sid

    zrow = jnp.zeros((L,), jnp.float32)

    @pl.loop(0, RPS)
    def _(i):
        stage_v[i, :] = zrow

    pltpu.sync_copy(stage_v, acc_sh.at[pl.ds(sid * RPS, RPS)])
    pltpu.sync_copy(table_hbm.at[pl.ds(sid * TRS, TRS)],
                    table_sh.at[pl.ds(sid * TRS, TRS)])
    pltpu.sync_copy(e_hbm.at[0, pl.ds(wid * CH_PER_W, CH_PER_W)], src_v)
    pltpu.sync_copy(e_hbm.at[1, pl.ds(wid * CH_PER_W, CH_PER_W)], dst_v)
    plsc.subcore_barrier()

    def gat(j, b):
        tbl = table_sh if b % 2 == 0 else table_hbm
        return pltpu.make_async_copy(
            tbl.at[src_v.at[j]], bufs[b], gsem[b])

    def scat(j, b):
        return pltpu.make_async_copy(bufs[b], acc_sh.at[dst_v.at[j]], ssem[b])

    for b in range(NBUF):
        gat(b, b).start()

    @pl.loop(0, CH_PER_W - NBUF, step=NBUF)
    def _(jj):
        for b in range(NBUF):
            gat(jj + b, b).wait()
            scat(jj + b, b).start(add=True)
        for b in range(NBUF):
            scat(jj + b, b).wait()
            gat(jj + NBUF + b, b).start()

    for b in range(NBUF):
        gat(CH_PER_W - NBUF + b, b).wait()
        scat(CH_PER_W - NBUF + b, b).start(add=True)
    for b in range(NBUF):
        scat(0, b).wait()

    plsc.subcore_barrier()
    pltpu.sync_copy(
        acc_sh.at[pl.ds(sid * RPS, RPS)],
        out_hbm.at[cid, pl.ds(sid * RPS, RPS)],
    )


def _mm1_body(xr_ref, w_ref, o_ref):
    o_ref[...] = jnp.dot(xr_ref[...], w_ref[...],
                         preferred_element_type=jnp.float32)


def _prep_body(h1p_ref, dp_ref, h1s_ref, dis_ref):
    deg = dp_ref[0, :PR, :] + dp_ref[1, :PR, :] + 1.0
    dis = lax.rsqrt(deg)
    dis_ref[...] = dis
    h1s_ref[...] = h1p_ref[...] * dis


def _post1_body(acc_ref, h1s_ref, dis_ref, b1_ref, o_ref):
    agg = acc_ref[0, :PR, :] + acc_ref[1, :PR, :] + h1s_ref[...]
    pre = agg * dis_ref[...] + b1_ref[...]
    o_ref[...] = jnp.maximum(pre, 0.0) * dis_ref[...]


def _final_body(acc_ref, o1s_ref, dis_ref, w2_ref, b2_ref, o_ref):
    hp = (acc_ref[0, :PR, :] + acc_ref[1, :PR, :] + o1s_ref[...]) * dis_ref[...]
    o_ref[...] = jnp.dot(
        hp, w2_ref[...], preferred_element_type=jnp.float32) + b2_ref[...]


def kernel(x, edge_index, W1, b1, W2, b2):
    e3 = jnp.concatenate(
        [edge_index.reshape(2, NW, E // NW), _PAD_EDGES],
        axis=2).reshape(2, EROWS, CHUNK)

    deg_parts = _sc_degree(e3)

    xr = jnp.pad(x, ((0, N_TBL - N), (0, 0))).reshape(PR, 8 * F_IN)
    w1bd = jnp.kron(jnp.eye(128 // H, dtype=W1.dtype), W1)
    h1p = pl.pallas_call(
        _mm1_body,
        out_shape=jax.ShapeDtypeStruct((PR, 128), jnp.float32),
    )(xr, w1bd)

    h1sp, disp = pl.pallas_call(
        _prep_body,
        out_shape=[
            jax.ShapeDtypeStruct((PR, 128), jnp.float32),
            jax.ShapeDtypeStruct((PR, 128), jnp.float32),
        ],
    )(h1p, deg_parts.reshape(NC, ACC_PR, 128))

    acc1 = _sc_aggregate(h1sp.reshape(N_TBL, H), e3)

    b1t = jnp.tile(b1, 128 // H).reshape(1, 128)
    o1sp = pl.pallas_call(
        _post1_body,
        out_shape=jax.ShapeDtypeStruct((PR, 128), jnp.float32),
    )(acc1.reshape(NC, ACC_PR, 128), h1sp, disp, b1t)

    acc2 = _sc_aggregate(o1sp.reshape(N_TBL, H), e3)

    w2bd = jnp.kron(jnp.eye(128 // H, dtype=W2.dtype), W2)
    b2t = jnp.tile(b2, 128 // H).reshape(1, 8 * C)
    out24 = pl.pallas_call(
        _final_body,
        out_shape=jax.ShapeDtypeStruct((PR, 8 * C), jnp.float32),
    )(acc2.reshape(NC, ACC_PR, 128), o1sp, disp, w2bd, b2t)

    return out24.reshape(N_TBL, C)[:N, :]
